# baseline (speedup 1.0000x reference)
import functools

import jax
import jax.numpy as jnp
from jax import lax
from jax.experimental import pallas as pl
from jax.experimental.pallas import tpu as pltpu
from jax.experimental.pallas import tpu_sc as plsc

T = 2048
D = 768
DFF = 2048
E = 8
K = 2
BM = 256
R_ROUTED = T * K + E * BM
NT_R = R_ROUTED // BM
NT_S = T // BM

NC, NS = 2, 16
NW = NC * NS
_MR = (T * K) // 128


def _gating(x, gate_w):
    logits = (x @ gate_w.T).astype(jnp.float32)
    lse = jax.scipy.special.logsumexp(logits, axis=-1)
    m1 = jnp.max(logits, axis=-1)
    a1 = jnp.argmax(logits, axis=-1)
    masked = jnp.where(jax.nn.one_hot(a1, E, dtype=jnp.bool_), -jnp.inf,
                       logits)
    m2 = jnp.max(masked, axis=-1)
    a2 = jnp.argmax(masked, axis=-1)
    gates = jnp.exp(jnp.stack([m1, m2], axis=0) - lse[None, :])
    ee = jnp.stack([a1, a2], axis=0).astype(jnp.int32)
    return ee, gates


def _meta_body(ee_ref, slot_ref, eid_ref):
    ee = ee_ref[...]
    ri = lax.broadcasted_iota(jnp.int32, (_MR, _MR), 0)
    rj = lax.broadcasted_iota(jnp.int32, (_MR, _MR), 1)
    tril = (rj < ri).astype(jnp.float32)
    ci = lax.broadcasted_iota(jnp.int32, (128, 128), 0)
    cj = lax.broadcasted_iota(jnp.int32, (128, 128), 1)
    triu = (ci < cj).astype(jnp.float32)
    slot_acc = jnp.zeros((_MR, 128), jnp.float32)
    rt = lax.broadcasted_iota(jnp.int32, (1, 32), 1).astype(jnp.float32) * BM
    eid_acc = jnp.zeros((1, 32), jnp.float32)
    start = jnp.float32(0.0)
    for e in range(E):
        oh = (ee == e).astype(jnp.float32)
        rowsum = jnp.sum(oh, axis=1, keepdims=True)
        prev = lax.dot_general(tril, rowsum, (((1,), (0,)), ((), ())),
                               preferred_element_type=jnp.float32)
        pref = lax.dot_general(oh, triu, (((1,), (0,)), ((), ())),
                               preferred_element_type=jnp.float32)
        slot_acc = slot_acc + oh * (start + prev + pref)
        cnt = jnp.sum(rowsum)
        start = start + jnp.ceil(cnt * (1.0 / BM)) * BM
        eid_acc = eid_acc + (start <= rt).astype(jnp.float32)
    slot_ref[...] = slot_acc.astype(jnp.int32)
    eid = jnp.minimum(eid_acc, E - 1)
    active = (rt < start).astype(jnp.float32)
    eid_ref[...] = jnp.concatenate([eid, active], axis=0).astype(jnp.int32)


def _metadata(ee):
    return pl.pallas_call(
        _meta_body,
        out_shape=(jax.ShapeDtypeStruct((_MR, 128), jnp.int32),
                   jax.ShapeDtypeStruct((2, 32), jnp.int32)),
    )(ee.reshape(_MR, 128))


_D_TOK = T // NW


def _dispatch_body(x_hbm, sidx_hbm, gg_hbm, xr_hbm, wsl_hbm,
                   xbuf, idx_v, gbuf, s0, s1, s2, s3, s4):
    wid = lax.axis_index("s") * NC + lax.axis_index("c")
    tbase = wid * _D_TOK
    row = wid // 2
    col = (wid % 2) * _D_TOK
    hx = pltpu.async_copy(x_hbm.at[pl.ds(tbase, _D_TOK)], xbuf, s0)
    hi0 = pltpu.async_copy(sidx_hbm.at[row, pl.ds(col, _D_TOK)],
                           idx_v.at[0], s1)
    hi1 = pltpu.async_copy(sidx_hbm.at[16 + row, pl.ds(col, _D_TOK)],
                           idx_v.at[1], s2)
    hg0 = pltpu.async_copy(gg_hbm.at[0, pl.ds(tbase, _D_TOK)], gbuf.at[0], s3)
    hg1 = pltpu.async_copy(gg_hbm.at[1, pl.ds(tbase, _D_TOK)], gbuf.at[1], s4)
    hx.wait()
    hi0.wait()
    hi1.wait()
    hg0.wait()
    hg1.wait()
    h0 = pltpu.async_copy(xbuf, xr_hbm.at[idx_v.at[0]], s0)
    h1 = pltpu.async_copy(xbuf, xr_hbm.at[idx_v.at[1]], s1)
    h2 = pltpu.async_copy(gbuf.at[0], wsl_hbm.at[idx_v.at[0]], s2)
    h3 = pltpu.async_copy(gbuf.at[1], wsl_hbm.at[idx_v.at[1]], s3)
    h0.wait()
    h1.wait()
    h2.wait()
    h3.wait()


def _dispatch(x, sidx, gg):
    return pl.kernel(
        _dispatch_body,
        out_type=(jax.ShapeDtypeStruct((R_ROUTED, D), jnp.float32),
                  jax.ShapeDtypeStruct((R_ROUTED,), jnp.float32)),
        mesh=plsc.VectorSubcoreMesh(core_axis_name="c", subcore_axis_name="s"),
        scratch_types=[
            pltpu.VMEM((_D_TOK, D), jnp.float32),
            pltpu.VMEM((K, _D_TOK), jnp.int32),
            pltpu.VMEM((K, _D_TOK), jnp.float32),
            pltpu.SemaphoreType.DMA,
            pltpu.SemaphoreType.DMA,
            pltpu.SemaphoreType.DMA,
            pltpu.SemaphoreType.DMA,
            pltpu.SemaphoreType.DMA,
        ],
    )(x, sidx, gg)


def _shared_body(x_ref, w1_ref, w3_ref, w2_ref, o_ref):
    x = x_ref[...]
    g = lax.dot_general(x, w1_ref[...], (((1,), (1,)), ((), ())),
                        preferred_element_type=jnp.float32)
    u = lax.dot_general(x, w3_ref[...], (((1,), (1,)), ((), ())),
                        preferred_element_type=jnp.float32)
    h = g * jax.nn.sigmoid(g) * u
    o_ref[...] = lax.dot_general(h, w2_ref[...], (((1,), (1,)), ((), ())),
                                 preferred_element_type=jnp.float32)


def _shared_ffn(x, w1s, w3s, w2s):
    return pl.pallas_call(
        _shared_body,
        grid=(NT_S,),
        in_specs=[
            pl.BlockSpec((BM, D), lambda i: (i, 0)),
            pl.BlockSpec((DFF, D), lambda i: (0, 0)),
            pl.BlockSpec((DFF, D), lambda i: (0, 0)),
            pl.BlockSpec((D, DFF), lambda i: (0, 0)),
        ],
        out_specs=pl.BlockSpec((BM, D), lambda i: (i, 0)),
        out_shape=jax.ShapeDtypeStruct((T, D), jnp.float32),
    )(x, w1s, w3s, w2s)


def _routed_body(s_ref, x_ref, w1_ref, w3_ref, w2_ref, sc_ref, o_ref):
    @pl.when(s_ref[1, pl.program_id(0)] == 1)
    def _():
        x = x_ref[...]
        g = lax.dot_general(x, w1_ref[0], (((1,), (1,)), ((), ())),
                            preferred_element_type=jnp.float32)
        u = lax.dot_general(x, w3_ref[0], (((1,), (1,)), ((), ())),
                            preferred_element_type=jnp.float32)
        h = g * jax.nn.sigmoid(g) * u
        y = lax.dot_general(h, w2_ref[0], (((1,), (1,)), ((), ())),
                            preferred_element_type=jnp.float32)
        o_ref[...] = y * sc_ref[0, :][:, None]


def _routed_ffn(tile_eid, xr, w1e, w3e, w2e, scales):
    def emap(i, s):
        return (s[0, i], 0, 0)

    grid_spec = pltpu.PrefetchScalarGridSpec(
        num_scalar_prefetch=1,
        grid=(NT_R,),
        in_specs=[
            pl.BlockSpec((BM, D), lambda i, s: (i, 0)),
            pl.BlockSpec((1, DFF, D), emap),
            pl.BlockSpec((1, DFF, D), emap),
            pl.BlockSpec((1, D, DFF), emap),
            pl.BlockSpec((1, BM), lambda i, s: (0, i)),
        ],
        out_specs=pl.BlockSpec((BM, D), lambda i, s: (i, 0)),
    )
    return pl.pallas_call(
        _routed_body,
        grid_spec=grid_spec,
        out_shape=jax.ShapeDtypeStruct((R_ROUTED, D), jnp.float32),
    )(tile_eid, xr, w1e, w3e, w2e, scales)


_C_TOK = T // NW
_C_SUB = 16
_C_CH = _C_TOK // _C_SUB
_DL = D // 16


def _combine_body(ys_hbm, yr_hbm, pos_hbm, out_hbm, acc_v, g0_v, g1_v, idx_v,
                  sa, si0, si1, sg0, sg1, sg2, sg3):
    wid = lax.axis_index("s") * NC + lax.axis_index("c")
    base = wid * _C_TOK
    row = wid // 2
    col = (wid % 2) * _C_TOK
    ha = pltpu.async_copy(ys_hbm.at[pl.ds(base, _C_TOK)], acc_v, sa)
    hi0 = pltpu.async_copy(pos_hbm.at[row, pl.ds(col, _C_TOK)],
                           idx_v.at[0], si0)
    hi1 = pltpu.async_copy(pos_hbm.at[16 + row, pl.ds(col, _C_TOK)],
                           idx_v.at[1], si1)
    hi0.wait()
    hi1.wait()

    bufs = (g0_v, g1_v)
    sems = ((sg0, sg1), (sg2, sg3))

    def issue(c):
        buf, sp = bufs[c % 2], sems[c % 2]
        sl = pl.ds(c * _C_SUB, _C_SUB)
        h0 = pltpu.async_copy(yr_hbm.at[idx_v.at[0, sl]],
                              buf.at[pl.ds(0, _C_SUB)], sp[0])
        h1 = pltpu.async_copy(yr_hbm.at[idx_v.at[1, sl]],
                              buf.at[pl.ds(_C_SUB, _C_SUB)], sp[1])
        return h0, h1

    hs = {0: issue(0)}
    ha.wait()
    for c in range(_C_CH):
        if c + 1 < _C_CH:
            hs[c + 1] = issue(c + 1)
        hs[c][0].wait()
        hs[c][1].wait()
        buf = bufs[c % 2]

        def row_add(r, carry, c=c, buf=buf):
            ar = c * _C_SUB + r
            for j in range(_DL):
                sl = pl.ds(j * 16, 16)
                acc_v[ar, sl] = acc_v[ar, sl] + buf[r, sl] + buf[_C_SUB + r, sl]
            return carry

        lax.fori_loop(0, _C_SUB, row_add, 0)
    pltpu.sync_copy(acc_v, out_hbm.at[pl.ds(base, _C_TOK)])


def _combine(ys, yr, pos):
    return pl.kernel(
        _combine_body,
        out_type=jax.ShapeDtypeStruct((T, D), jnp.float32),
        mesh=plsc.VectorSubcoreMesh(core_axis_name="c", subcore_axis_name="s"),
        scratch_types=[
            pltpu.VMEM((_C_TOK, D), jnp.float32),
            pltpu.VMEM((2 * _C_SUB, D), jnp.float32),
            pltpu.VMEM((2 * _C_SUB, D), jnp.float32),
            pltpu.VMEM((K, _C_TOK), jnp.int32),
            pltpu.SemaphoreType.DMA,
            pltpu.SemaphoreType.DMA,
            pltpu.SemaphoreType.DMA,
            pltpu.SemaphoreType.DMA,
            pltpu.SemaphoreType.DMA,
            pltpu.SemaphoreType.DMA,
            pltpu.SemaphoreType.DMA,
        ],
    )(ys, yr, pos)


def kernel(hidden_states, gate_w, shared_w1, shared_w2, shared_w3,
           exp_w1, exp_w2, exp_w3):
    x = hidden_states.reshape(T, D)
    ee, gates = _gating(x, gate_w)
    slot2d, eid2d = _metadata(ee)

    w1s = shared_w1.reshape(DFF, D)
    w3s = shared_w3.reshape(DFF, D)
    w2s = (shared_w2 * 0.125).transpose(1, 0, 2).reshape(D, DFF)

    xr, wsl = _dispatch(x, slot2d, gates)
    ys = _shared_ffn(x, w1s, w3s, w2s)
    yr = _routed_ffn(eid2d, xr, exp_w1, exp_w3, exp_w2,
                     wsl.reshape(1, R_ROUTED))
    out = _combine(ys, yr, slot2d)
    return out.reshape(1, T, D)


# speedup vs baseline: 1.0167x; 1.0167x over previous
import functools

import jax
import jax.numpy as jnp
from jax import lax
from jax.experimental import pallas as pl
from jax.experimental.pallas import tpu as pltpu
from jax.experimental.pallas import tpu_sc as plsc

T = 2048
D = 768
DFF = 2048
E = 8
K = 2
BM = 256
R_ROUTED = T * K + E * BM
NT_R = R_ROUTED // BM
NT_S = T // BM

NC, NS = 2, 16
NW = NC * NS
_MR = (T * K) // 128


def _gating(x, gate_w):
    logits = (x @ gate_w.T).astype(jnp.float32)
    lse = jax.scipy.special.logsumexp(logits, axis=-1)
    m1 = jnp.max(logits, axis=-1)
    a1 = jnp.argmax(logits, axis=-1)
    masked = jnp.where(jax.nn.one_hot(a1, E, dtype=jnp.bool_), -jnp.inf,
                       logits)
    m2 = jnp.max(masked, axis=-1)
    a2 = jnp.argmax(masked, axis=-1)
    gates = jnp.exp(jnp.stack([m1, m2], axis=0) - lse[None, :])
    ee = jnp.stack([a1, a2], axis=0).astype(jnp.int32)
    return ee, gates


def _meta_body(ee_ref, slot_ref, eid_ref):
    ee = ee_ref[...]
    ri = lax.broadcasted_iota(jnp.int32, (_MR, _MR), 0)
    rj = lax.broadcasted_iota(jnp.int32, (_MR, _MR), 1)
    tril = (rj < ri).astype(jnp.float32)
    ci = lax.broadcasted_iota(jnp.int32, (128, 128), 0)
    cj = lax.broadcasted_iota(jnp.int32, (128, 128), 1)
    triu = (ci < cj).astype(jnp.float32)
    slot_acc = jnp.zeros((_MR, 128), jnp.float32)
    rt = lax.broadcasted_iota(jnp.int32, (1, 32), 1).astype(jnp.float32) * BM
    eid_acc = jnp.zeros((1, 32), jnp.float32)
    start = jnp.float32(0.0)
    for e in range(E):
        oh = (ee == e).astype(jnp.float32)
        rowsum = jnp.sum(oh, axis=1, keepdims=True)
        prev = lax.dot_general(tril, rowsum, (((1,), (0,)), ((), ())),
                               preferred_element_type=jnp.float32)
        pref = lax.dot_general(oh, triu, (((1,), (0,)), ((), ())),
                               preferred_element_type=jnp.float32)
        slot_acc = slot_acc + oh * (start + prev + pref)
        cnt = jnp.sum(rowsum)
        start = start + jnp.ceil(cnt * (1.0 / BM)) * BM
        eid_acc = eid_acc + (start <= rt).astype(jnp.float32)
    slot_ref[...] = slot_acc.astype(jnp.int32)
    eid = jnp.minimum(eid_acc, E - 1)
    active = (rt < start).astype(jnp.float32)
    eid_ref[...] = jnp.concatenate([eid, active], axis=0).astype(jnp.int32)


def _metadata(ee):
    return pl.pallas_call(
        _meta_body,
        out_shape=(jax.ShapeDtypeStruct((_MR, 128), jnp.int32),
                   jax.ShapeDtypeStruct((2, 32), jnp.int32)),
    )(ee.reshape(_MR, 128))


_D_TOK = T // NW


def _dispatch_body(x_hbm, sidx_hbm, gg_hbm, xr_hbm, wsl_hbm,
                   xbuf, idx_v, gbuf, s0, s1, s2, s3, s4):
    wid = lax.axis_index("s") * NC + lax.axis_index("c")
    tbase = wid * _D_TOK
    hx = pltpu.async_copy(x_hbm.at[pl.ds(tbase, _D_TOK)], xbuf, s0)
    hi0 = pltpu.async_copy(sidx_hbm.at[0, wid], idx_v.at[0], s1)
    hi1 = pltpu.async_copy(sidx_hbm.at[1, wid], idx_v.at[1], s2)
    hg0 = pltpu.async_copy(gg_hbm.at[0, wid], gbuf.at[0], s3)
    hg1 = pltpu.async_copy(gg_hbm.at[1, wid], gbuf.at[1], s4)
    hx.wait()
    hi0.wait()
    hi1.wait()
    hg0.wait()
    hg1.wait()
    h0 = pltpu.async_copy(xbuf, xr_hbm.at[idx_v.at[0]], s0)
    h1 = pltpu.async_copy(xbuf, xr_hbm.at[idx_v.at[1]], s1)
    h2 = pltpu.async_copy(gbuf.at[0], wsl_hbm.at[idx_v.at[0]], s2)
    h3 = pltpu.async_copy(gbuf.at[1], wsl_hbm.at[idx_v.at[1]], s3)
    h0.wait()
    h1.wait()
    h2.wait()
    h3.wait()


def _dispatch(x, sidx, gg):
    return pl.kernel(
        _dispatch_body,
        out_type=(jax.ShapeDtypeStruct((R_ROUTED, D), jnp.float32),
                  jax.ShapeDtypeStruct((R_ROUTED,), jnp.float32)),
        mesh=plsc.VectorSubcoreMesh(core_axis_name="c", subcore_axis_name="s"),
        scratch_types=[
            pltpu.VMEM((_D_TOK, D), jnp.float32),
            pltpu.VMEM((K, _D_TOK), jnp.int32),
            pltpu.VMEM((K, _D_TOK), jnp.float32),
            pltpu.SemaphoreType.DMA,
            pltpu.SemaphoreType.DMA,
            pltpu.SemaphoreType.DMA,
            pltpu.SemaphoreType.DMA,
            pltpu.SemaphoreType.DMA,
        ],
    )(x, sidx, gg)


def _shared_body(x_ref, w1_ref, w3_ref, w2_ref, o_ref):
    x = x_ref[...]
    g = lax.dot_general(x, w1_ref[...], (((1,), (1,)), ((), ())),
                        preferred_element_type=jnp.float32)
    u = lax.dot_general(x, w3_ref[...], (((1,), (1,)), ((), ())),
                        preferred_element_type=jnp.float32)
    h = g * jax.nn.sigmoid(g) * u
    o_ref[...] = lax.dot_general(h, w2_ref[...], (((1,), (1,)), ((), ())),
                                 preferred_element_type=jnp.float32)


def _shared_ffn(x, w1s, w3s, w2s):
    return pl.pallas_call(
        _shared_body,
        grid=(NT_S,),
        in_specs=[
            pl.BlockSpec((BM, D), lambda i: (i, 0)),
            pl.BlockSpec((DFF, D), lambda i: (0, 0)),
            pl.BlockSpec((DFF, D), lambda i: (0, 0)),
            pl.BlockSpec((D, DFF), lambda i: (0, 0)),
        ],
        out_specs=pl.BlockSpec((BM, D), lambda i: (i, 0)),
        out_shape=jax.ShapeDtypeStruct((T, D), jnp.float32),
    )(x, w1s, w3s, w2s)


def _routed_body(s_ref, x_ref, w1_ref, w3_ref, w2_ref, sc_ref, o_ref):
    @pl.when(s_ref[1, pl.program_id(0)] == 1)
    def _():
        x = x_ref[...]
        g = lax.dot_general(x, w1_ref[0], (((1,), (1,)), ((), ())),
                            preferred_element_type=jnp.float32)
        u = lax.dot_general(x, w3_ref[0], (((1,), (1,)), ((), ())),
                            preferred_element_type=jnp.float32)
        h = g * jax.nn.sigmoid(g) * u
        y = lax.dot_general(h, w2_ref[0], (((1,), (1,)), ((), ())),
                            preferred_element_type=jnp.float32)
        o_ref[...] = y * sc_ref[0, 0, :][:, None]


def _routed_ffn(tile_eid, xr, w1e, w3e, w2e, scales):
    def emap(i, s):
        return (s[0, i], 0, 0)

    grid_spec = pltpu.PrefetchScalarGridSpec(
        num_scalar_prefetch=1,
        grid=(NT_R,),
        in_specs=[
            pl.BlockSpec((BM, D), lambda i, s: (i, 0)),
            pl.BlockSpec((1, DFF, D), emap),
            pl.BlockSpec((1, DFF, D), emap),
            pl.BlockSpec((1, D, DFF), emap),
            pl.BlockSpec((1, 1, BM), lambda i, s: (i, 0, 0)),
        ],
        out_specs=pl.BlockSpec((BM, D), lambda i, s: (i, 0)),
    )
    return pl.pallas_call(
        _routed_body,
        grid_spec=grid_spec,
        out_shape=jax.ShapeDtypeStruct((R_ROUTED, D), jnp.float32),
    )(tile_eid, xr, w1e, w3e, w2e, scales)


_C_TOK = T // NW
_C_SUB = 16
_C_CH = _C_TOK // _C_SUB
_DL = D // 16


def _combine_body(ys_hbm, yr_hbm, pos_hbm, out_hbm, acc_v, g0_v, g1_v, idx_v,
                  sa, si0, si1, sg0, sg1, sg2, sg3):
    wid = lax.axis_index("s") * NC + lax.axis_index("c")
    base = wid * _C_TOK
    ha = pltpu.async_copy(ys_hbm.at[pl.ds(base, _C_TOK)], acc_v, sa)
    hi0 = pltpu.async_copy(pos_hbm.at[pl.ds(base, _C_TOK)], idx_v.at[0], si0)
    hi1 = pltpu.async_copy(pos_hbm.at[pl.ds(T + base, _C_TOK)], idx_v.at[1],
                           si1)
    hi0.wait()
    hi1.wait()

    bufs = (g0_v, g1_v)
    sems = ((sg0, sg1), (sg2, sg3))

    def issue(c):
        buf, sp = bufs[c % 2], sems[c % 2]
        sl = pl.ds(c * _C_SUB, _C_SUB)
        h0 = pltpu.async_copy(yr_hbm.at[idx_v.at[0, sl]],
                              buf.at[pl.ds(0, _C_SUB)], sp[0])
        h1 = pltpu.async_copy(yr_hbm.at[idx_v.at[1, sl]],
                              buf.at[pl.ds(_C_SUB, _C_SUB)], sp[1])
        return h0, h1

    hs = {0: issue(0)}
    ha.wait()
    for c in range(_C_CH):
        if c + 1 < _C_CH:
            hs[c + 1] = issue(c + 1)
        hs[c][0].wait()
        hs[c][1].wait()
        buf = bufs[c % 2]

        def row_add(r, carry, c=c, buf=buf):
            ar = c * _C_SUB + r
            for j in range(_DL):
                sl = pl.ds(j * 16, 16)
                acc_v[ar, sl] = acc_v[ar, sl] + buf[r, sl] + buf[_C_SUB + r, sl]
            return carry

        lax.fori_loop(0, _C_SUB, row_add, 0)
    pltpu.sync_copy(acc_v, out_hbm.at[pl.ds(base, _C_TOK)])


def _combine(ys, yr, pos):
    return pl.kernel(
        _combine_body,
        out_type=jax.ShapeDtypeStruct((T, D), jnp.float32),
        mesh=plsc.VectorSubcoreMesh(core_axis_name="c", subcore_axis_name="s"),
        scratch_types=[
            pltpu.VMEM((_C_TOK, D), jnp.float32),
            pltpu.VMEM((2 * _C_SUB, D), jnp.float32),
            pltpu.VMEM((2 * _C_SUB, D), jnp.float32),
            pltpu.VMEM((K, _C_TOK), jnp.int32),
            pltpu.SemaphoreType.DMA,
            pltpu.SemaphoreType.DMA,
            pltpu.SemaphoreType.DMA,
            pltpu.SemaphoreType.DMA,
            pltpu.SemaphoreType.DMA,
            pltpu.SemaphoreType.DMA,
            pltpu.SemaphoreType.DMA,
        ],
    )(ys, yr, pos)


def kernel(hidden_states, gate_w, shared_w1, shared_w2, shared_w3,
           exp_w1, exp_w2, exp_w3):
    x = hidden_states.reshape(T, D)
    ee, gates = _gating(x, gate_w)
    slot2d, eid2d = _metadata(ee)
    slot = slot2d.reshape(K * T)
    sidx = slot.reshape(K, NW, _D_TOK)
    ggw = gates.reshape(K, NW, _D_TOK)

    w1s = shared_w1.reshape(DFF, D)
    w3s = shared_w3.reshape(DFF, D)
    w2s = (shared_w2 * 0.125).transpose(1, 0, 2).reshape(D, DFF)

    xr, wsl = _dispatch(x, sidx, ggw)
    ys = _shared_ffn(x, w1s, w3s, w2s)
    yr = _routed_ffn(eid2d, xr, exp_w1, exp_w3, exp_w2,
                     wsl.reshape(NT_R, 1, BM))
    out = _combine(ys, yr, slot)
    return out.reshape(1, T, D)
